# baseline (speedup 1.0000x reference)
import jax
import jax.numpy as jnp
from jax import lax
from jax.experimental import pallas as pl
from jax.experimental.pallas import tpu as pltpu
from jax.experimental.pallas import tpu_sc as plsc

B = 16384
D = 16
V = 100000

_INFO = plsc.get_sparse_core_info()
_NC = _INFO.num_cores
_NS = _INFO.num_subcores
_NW = _NC * _NS
_BPW = B // _NW


def _body(contrib_table, recip_table, contrib_idx, recip_idx,
          xr_out, xc_out,
          idx_rv, idx_cv, out_r, out_c,
          sem_wr, sem_wc):
    wid = lax.axis_index("s") * _NC + lax.axis_index("c")
    base = wid * _BPW
    sl = pl.ds(base, _BPW)
    pltpu.sync_copy(recip_idx.at[sl], idx_rv)
    pltpu.sync_copy(contrib_idx.at[sl], idx_cv)

    half = _BPW // 2

    def make_grp(h):
        def grp(g, _):
            j0 = h * half + g * 16
            o0 = g * 16
            vr = idx_rv[pl.ds(j0, 16)]
            vc = idx_cv[pl.ds(j0, 16)]
            for l in range(16):
                pltpu.sync_copy(recip_table.at[pl.ds(vr[l], 1)],
                                out_r.at[pl.ds(o0 + l, 1)])
                pltpu.sync_copy(contrib_table.at[pl.ds(vc[l], 1)],
                                out_c.at[pl.ds(o0 + l, 1)])
            return 0
        return grp

    wr = wc = None
    for h in range(2):
        if wr is not None:
            wr.wait()
            wc.wait()
        lax.fori_loop(0, half // 16, make_grp(h), 0)
        hs = pl.ds(base + h * half, half)
        wr = pltpu.async_copy(out_r, xr_out.at[hs], sem_wr)
        wc = pltpu.async_copy(out_c, xc_out.at[hs], sem_wc)
    wr.wait()
    wc.wait()


@jax.jit
def kernel(contrib_table, recip_table, contrib_idx, recip_idx):
    mesh = plsc.VectorSubcoreMesh(core_axis_name="c", subcore_axis_name="s")
    xr, xc = pl.kernel(
        _body,
        mesh=mesh,
        out_type=(
            jax.ShapeDtypeStruct((B, D), jnp.float32),
            jax.ShapeDtypeStruct((B, D), jnp.float32),
        ),
        scratch_types=[
            pltpu.VMEM((_BPW,), jnp.int32),
            pltpu.VMEM((_BPW,), jnp.int32),
            pltpu.VMEM((_BPW // 2, D), jnp.float32),
            pltpu.VMEM((_BPW // 2, D), jnp.float32),
            pltpu.SemaphoreType.DMA,
            pltpu.SemaphoreType.DMA,
        ],
        compiler_params=pltpu.CompilerParams(needs_layout_passes=False),
    )(contrib_table, recip_table, contrib_idx, recip_idx)
    return xr, xc


# speedup vs baseline: 5.8699x; 5.8699x over previous
import jax
import jax.numpy as jnp
from jax import lax
from jax.experimental import pallas as pl
from jax.experimental.pallas import tpu as pltpu
from jax.experimental.pallas import tpu_sc as plsc

B = 16384
D = 16
V = 100000

_INFO = plsc.get_sparse_core_info()
_NC = _INFO.num_cores
_NS = _INFO.num_subcores
_NW = _NC * _NS
_BPW = B // _NW


def _body(contrib_table, recip_table, contrib_idx, recip_idx,
          xr_out, xc_out,
          idx_rv, idx_cv, out_r, out_c,
          sem_gr, sem_gc, sem_wr, sem_wc):
    wid = lax.axis_index("s") * _NC + lax.axis_index("c")
    base = wid * _BPW
    sl = pl.ds(base, _BPW)
    pltpu.sync_copy(recip_idx.at[sl], idx_rv)
    pltpu.sync_copy(contrib_idx.at[sl], idx_cv)

    half = _BPW // 2

    def make_grp(h):
        def grp(g, _):
            j0 = h * half + g * 16
            o0 = g * 16
            vr = idx_rv[pl.ds(j0, 16)]
            vc = idx_cv[pl.ds(j0, 16)]
            for l in range(16):
                pltpu.async_copy(recip_table.at[pl.ds(vr[l], 1)],
                                 out_r.at[pl.ds(o0 + l, 1)], sem_gr)
                pltpu.async_copy(contrib_table.at[pl.ds(vc[l], 1)],
                                 out_c.at[pl.ds(o0 + l, 1)], sem_gc)
            return 0
        return grp

    wr = wc = None
    for h in range(2):
        if wr is not None:
            wr.wait()
            wc.wait()
        lax.fori_loop(0, half // 16, make_grp(h), 0)
        pltpu.make_async_copy(xr_out.at[pl.ds(base, half)],
                              out_r, sem_gr).wait()
        pltpu.make_async_copy(xc_out.at[pl.ds(base, half)],
                              out_c, sem_gc).wait()
        hs = pl.ds(base + h * half, half)
        wr = pltpu.async_copy(out_r, xr_out.at[hs], sem_wr)
        wc = pltpu.async_copy(out_c, xc_out.at[hs], sem_wc)
    wr.wait()
    wc.wait()


@jax.jit
def kernel(contrib_table, recip_table, contrib_idx, recip_idx):
    mesh = plsc.VectorSubcoreMesh(core_axis_name="c", subcore_axis_name="s")
    xr, xc = pl.kernel(
        _body,
        mesh=mesh,
        out_type=(
            jax.ShapeDtypeStruct((B, D), jnp.float32),
            jax.ShapeDtypeStruct((B, D), jnp.float32),
        ),
        scratch_types=[
            pltpu.VMEM((_BPW,), jnp.int32),
            pltpu.VMEM((_BPW,), jnp.int32),
            pltpu.VMEM((_BPW // 2, D), jnp.float32),
            pltpu.VMEM((_BPW // 2, D), jnp.float32),
            pltpu.SemaphoreType.DMA,
            pltpu.SemaphoreType.DMA,
            pltpu.SemaphoreType.DMA,
            pltpu.SemaphoreType.DMA,
        ],
        compiler_params=pltpu.CompilerParams(needs_layout_passes=False),
    )(contrib_table, recip_table, contrib_idx, recip_idx)
    return xr, xc
